# baseline (speedup 1.0000x reference)
import functools

import jax
import jax.numpy as jnp
from jax import lax
from jax.experimental import pallas as pl
from jax.experimental.pallas import tpu as pltpu
from jax.experimental.pallas import tpu_sc as plsc

N = 10000
D = 128
R = 16
E = 320000
NEG = 0.2
NC = 2
NS = 16
NW = NC * NS
ET = E // NW
ET_PAD = 10624
CHUNK = 256
LOGC = 8
C = 40
N_PAD = CHUNK * C
NB = CHUNK * R
DUMP = NB
CPS = C // NC
B = 16

_mesh = plsc.VectorSubcoreMesh(core_axis_name="c", subcore_axis_name="s",
                               num_cores=NC, num_subcores=NS)


def _pre_body(src_h, dst_h, typ_h, nid_h, emb_h,
              psrc_h, pbid_h, poff_h, h0_h,
              vsrc, vdst, vtyp, vbid, vg, osrc, obid,
              offv, idx64, rows64, sem):
    c = lax.axis_index("c")
    s = lax.axis_index("s")
    t = c * NS + s

    base = pl.multiple_of(t * ET, 8)
    pltpu.sync_copy(src_h.at[pl.ds(base, ET)], vsrc)
    pltpu.sync_copy(dst_h.at[pl.ds(base, ET)], vdst)
    pltpu.sync_copy(typ_h.at[pl.ds(base, ET)], vtyp)

    def cbody(i, _):
        d16 = vdst[pl.ds(i * 16, 16)]
        t16 = vtyp[pl.ds(i * 16, 16)]
        g16 = lax.shift_right_logical(d16, LOGC)
        b16 = (d16 & jnp.int32(CHUNK - 1)) * R + t16
        vg[pl.ds(i * 16, 16)] = g16
        vbid[pl.ds(i * 16, 16)] = b16
        return 0
    lax.fori_loop(0, ET // 16, cbody, 0)

    def ibody(i, _):
        osrc[pl.ds(i * 16, 16)] = jnp.zeros((16,), jnp.int32)
        obid[pl.ds(i * 16, 16)] = jnp.full((16,), DUMP, jnp.int32)
        return 0
    lax.fori_loop(0, (ET_PAD + 16) // 16, ibody, 0)

    lanes = lax.iota(jnp.int32, 16)
    starts = [jnp.zeros((16,), jnp.int32) for _ in range(3)]
    pos = jnp.int32(0)
    for ci in range(C):
        pv = jnp.full((16,), pos, jnp.int32)
        w = ci // 16
        starts[w] = jnp.where(lanes == ci - w * 16, pv, starts[w])

        def gbody(i, p):
            g16 = vg[pl.ds(i * 16, 16)]
            m = g16 == ci
            mi = m.astype(jnp.int32)
            rank = plsc.cumsum(mi) - mi
            tgt = jnp.where(m, p + rank, ET_PAD + lanes)
            plsc.store_scatter(osrc, [tgt], vsrc[pl.ds(i * 16, 16)])
            plsc.store_scatter(obid, [tgt], vbid[pl.ds(i * 16, 16)])
            return p + plsc.all_reduce_population_count(m)[0]
        pos = lax.fori_loop(0, ET // 16, gbody, pos)
        pos = (pos + B - 1) & jnp.int32(~(B - 1))
    starts[2] = jnp.where(lanes >= C - 32, jnp.full((16,), pos, jnp.int32),
                          starts[2])
    offv[pl.ds(0, 16)] = starts[0]
    offv[pl.ds(16, 16)] = starts[1]
    offv[pl.ds(32, 16)] = starts[2]

    tpad = pl.multiple_of(t * ET_PAD, 8)
    pltpu.sync_copy(osrc.at[pl.ds(0, ET_PAD)], psrc_h.at[pl.ds(tpad, ET_PAD)])
    pltpu.sync_copy(obid.at[pl.ds(0, ET_PAD)], pbid_h.at[pl.ds(tpad, ET_PAD)])
    pltpu.sync_copy(offv, poff_h.at[t])

    for q in range(5):
        r0 = pl.multiple_of(t * 320 + q * 64, 8)
        pltpu.sync_copy(nid_h.at[pl.ds(r0, 64)], idx64)
        pltpu.async_copy(emb_h.at[idx64], rows64, sem).wait()
        pltpu.sync_copy(rows64, h0_h.at[pl.ds(r0, 64), :])


@functools.partial(
    pl.kernel,
    out_type=(
        jax.ShapeDtypeStruct((NW * ET_PAD,), jnp.int32),
        jax.ShapeDtypeStruct((NW * ET_PAD,), jnp.int32),
        jax.ShapeDtypeStruct((NW, 48), jnp.int32),
        jax.ShapeDtypeStruct((N_PAD, D), jnp.float32),
    ),
    mesh=_mesh,
    compiler_params=pltpu.CompilerParams(needs_layout_passes=False),
    scratch_types=[
        pltpu.VMEM((ET,), jnp.int32),
        pltpu.VMEM((ET,), jnp.int32),
        pltpu.VMEM((ET,), jnp.int32),
        pltpu.VMEM((ET,), jnp.int32),
        pltpu.VMEM((ET,), jnp.int32),
        pltpu.VMEM((ET_PAD + 16,), jnp.int32),
        pltpu.VMEM((ET_PAD + 16,), jnp.int32),
        pltpu.VMEM((48,), jnp.int32),
        pltpu.VMEM((64,), jnp.int32),
        pltpu.VMEM((64, D), jnp.float32),
        pltpu.SemaphoreType.DMA,
    ],
)
def _preprocess(*args):
    _pre_body(*args)


def _layer_body(h_h, psrc_h, pbid_h, poff_h, s2_h,
                srcb, bidb, rows0, zbuf,
                offra, offrb, lcnt, cbuf, inv, semg0,
                acc, slab):
    c = lax.axis_index("c")
    s = lax.axis_index("s")
    z16f = jnp.zeros((16,), jnp.float32)
    o16f = jnp.ones((16,), jnp.float32)

    def zb(i, _):
        for k in range(8):
            zbuf[i, pl.ds(k * 16, 16)] = z16f
        return 0
    lax.fori_loop(0, 64, zb, 0)
    pltpu.sync_copy(poff_h.at[2 * s], offra)
    pltpu.sync_copy(poff_h.at[2 * s + 1], offrb)

    def chunk_body(cl, _):
        ci = c * CPS + cl
        for j in range(CHUNK // 64):
            pltpu.sync_copy(zbuf, acc.at[pl.ds(s * CHUNK + j * 64, 64), :])
        def zl_(i, _):
            lcnt[pl.ds(i * 16, 16)] = z16f
            return 0
        lax.fori_loop(0, (NB + 16) // 16, zl_, 0)
        plsc.subcore_barrier()

        for u in range(2):
            offr = offra if u == 0 else offrb
            tt = 2 * s + u
            st = plsc.load_gather(offr, [jnp.full((16,), ci, jnp.int32)])[0]
            en = plsc.load_gather(offr,
                                  [jnp.full((16,), ci + 1, jnp.int32)])[0]
            base = tt * ET_PAD + st
            nb = lax.shift_right_logical(en - st, 4)

            def bbody(bi, _):
                off = pl.multiple_of(base + bi * B, 8)
                pltpu.sync_copy(psrc_h.at[pl.ds(off, B)], srcb)
                pltpu.sync_copy(pbid_h.at[pl.ds(off, B)], bidb)
                pltpu.async_copy(h_h.at[srcb], rows0, semg0).wait()
                pltpu.sync_copy(rows0, acc.at[bidb], add=True)
                for g in range(B // 16):
                    idx16 = bidb[pl.ds(g * 16, 16)]
                    plsc.addupdate_scatter(lcnt, [idx16], o16f)
                return 0
            lax.fori_loop(0, nb, bbody, 0)

        pltpu.sync_copy(lcnt.at[pl.ds(0, NB)], slab.at[s])
        plsc.subcore_barrier()
        pltpu.sync_copy(slab.at[:, pl.ds(s * CHUNK, CHUNK)], cbuf)

        def nvbody(j, _):
            tot = z16f
            for i in range(NS):
                tot = tot + cbuf[i, pl.ds(j * 16, 16)]
            inv[pl.ds(j * 16, 16)] = 1.0 / jnp.maximum(tot, 1.0)
            return 0
        lax.fori_loop(0, CHUNK // 16, nvbody, 0)

        def fbody(fb, _):
            pltpu.sync_copy(acc.at[pl.ds(s * CHUNK + fb * B, B), :], rows0)

            def rowg(gi, _):
                iv = inv[pl.ds(fb * B + gi * 16, 16)]
                for j2 in range(16):
                    sv = iv[j2]
                    for k in range(8):
                        r = gi * 16 + j2
                        rows0[r, pl.ds(k * 16, 16)] = \
                            rows0[r, pl.ds(k * 16, 16)] * sv
                return 0
            lax.fori_loop(0, B // 16, rowg, 0)
            dst_row = ci * NB + s * CHUNK + fb * B
            pltpu.sync_copy(rows0, s2_h.at[pl.ds(dst_row, B), :])
            return 0
        lax.fori_loop(0, CHUNK // B, fbody, 0)
        plsc.subcore_barrier()
        return 0
    lax.fori_loop(0, CPS, chunk_body, 0)


@functools.partial(
    pl.kernel,
    out_type=jax.ShapeDtypeStruct((N_PAD * R, D), jnp.float32),
    mesh=_mesh,
    compiler_params=pltpu.CompilerParams(needs_layout_passes=False),
    scratch_types=[
        pltpu.VMEM((B,), jnp.int32),
        pltpu.VMEM((B,), jnp.int32),
        pltpu.VMEM((B, D), jnp.float32),
        pltpu.VMEM((64, D), jnp.float32),
        pltpu.VMEM((48,), jnp.int32),
        pltpu.VMEM((48,), jnp.int32),
        pltpu.VMEM((NB + 16,), jnp.float32),
        pltpu.VMEM((NS, CHUNK), jnp.float32),
        pltpu.VMEM((CHUNK,), jnp.float32),
        pltpu.SemaphoreType.DMA,
        pltpu.VMEM_SHARED((NB + 8, D), jnp.float32),
        pltpu.VMEM_SHARED((NS, NB), jnp.float32),
    ],
)
def _sc_layer(*args):
    _layer_body(*args)


BN = 256


def _tc_body(act, s3_ref, h_ref, wrel_ref, wself_ref, b_ref, out_ref):
    x3 = s3_ref[:]
    acc = jnp.dot(h_ref[:], wself_ref[:], preferred_element_type=jnp.float32)
    acc = acc + b_ref[:]
    for r in range(R):
        acc = acc + jnp.dot(x3[:, r, :], wrel_ref[r],
                            preferred_element_type=jnp.float32)
    if act:
        acc = jnp.maximum(acc, NEG * acc)
    out_ref[:] = acc


def _tc_layer(s3, h, wrel, wself, bb, act):
    return pl.pallas_call(
        functools.partial(_tc_body, act),
        grid=(N_PAD // BN,),
        in_specs=[
            pl.BlockSpec((BN, R, D), lambda i: (i, 0, 0)),
            pl.BlockSpec((BN, D), lambda i: (i, 0)),
            pl.BlockSpec((R, D, D), lambda i: (0, 0, 0)),
            pl.BlockSpec((D, D), lambda i: (0, 0)),
            pl.BlockSpec((1, D), lambda i: (0, 0)),
        ],
        out_specs=pl.BlockSpec((BN, D), lambda i: (i, 0)),
        out_shape=jax.ShapeDtypeStruct((N_PAD, D), jnp.float32),
    )(s3, h, wrel, wself, bb)


def kernel(emb_table, W_rel, W_self, b, edge_index, edge_type, node_id_copy):
    src = edge_index[0].astype(jnp.int32)
    dst = edge_index[1].astype(jnp.int32)
    typ = edge_type.astype(jnp.int32)
    nid = node_id_copy.astype(jnp.int32)
    nid_pad = jnp.concatenate([nid, jnp.zeros((N_PAD - N,), jnp.int32)])

    psrc, pbid, poff, h = _preprocess(src, dst, typ, nid_pad, emb_table)
    for l in range(3):
        s2 = _sc_layer(h, psrc, pbid, poff)
        s3 = s2.reshape(N_PAD, R, D)
        h = _tc_layer(s3, h, W_rel[l], W_self[l], b[l].reshape(1, D),
                      act=(l < 2))
    return h[:N]


# speedup vs baseline: 1.1231x; 1.1231x over previous
import functools

import jax
import jax.numpy as jnp
from jax import lax
from jax.experimental import pallas as pl
from jax.experimental.pallas import tpu as pltpu
from jax.experimental.pallas import tpu_sc as plsc

N = 10000
D = 128
R = 16
E = 320000
NEG = 0.2
NC = 2
NS = 16
NW = NC * NS
ET = E // NW
ET_PAD = 11264
CHUNK = 256
LOGC = 8
C = 40
N_PAD = CHUNK * C
NB = CHUNK * R
DUMP = NB
CPS = C // NC
B = 32

_mesh = plsc.VectorSubcoreMesh(core_axis_name="c", subcore_axis_name="s",
                               num_cores=NC, num_subcores=NS)


def _pre_body(src_h, dst_h, typ_h, nid_h, emb_h,
              psrc_h, pbid_h, poff_h, h0_h,
              vsrc, vdst, vtyp, vbid, vg, osrc, obid,
              offv, idx64, rows64, sem):
    c = lax.axis_index("c")
    s = lax.axis_index("s")
    t = c * NS + s

    base = pl.multiple_of(t * ET, 8)
    pltpu.sync_copy(src_h.at[pl.ds(base, ET)], vsrc)
    pltpu.sync_copy(dst_h.at[pl.ds(base, ET)], vdst)
    pltpu.sync_copy(typ_h.at[pl.ds(base, ET)], vtyp)

    def cbody(i, _):
        d16 = vdst[pl.ds(i * 16, 16)]
        t16 = vtyp[pl.ds(i * 16, 16)]
        g16 = lax.shift_right_logical(d16, LOGC)
        b16 = (d16 & jnp.int32(CHUNK - 1)) * R + t16
        vg[pl.ds(i * 16, 16)] = g16
        vbid[pl.ds(i * 16, 16)] = b16
        return 0
    lax.fori_loop(0, ET // 16, cbody, 0)

    def ibody(i, _):
        osrc[pl.ds(i * 16, 16)] = jnp.zeros((16,), jnp.int32)
        obid[pl.ds(i * 16, 16)] = jnp.full((16,), DUMP, jnp.int32)
        return 0
    lax.fori_loop(0, (ET_PAD + 16) // 16, ibody, 0)

    lanes = lax.iota(jnp.int32, 16)
    starts = [jnp.zeros((16,), jnp.int32) for _ in range(3)]
    pos = jnp.int32(0)
    for ci in range(C):
        pv = jnp.full((16,), pos, jnp.int32)
        w = ci // 16
        starts[w] = jnp.where(lanes == ci - w * 16, pv, starts[w])

        def gbody(i, p):
            g16 = vg[pl.ds(i * 16, 16)]
            m = g16 == ci
            mi = m.astype(jnp.int32)
            rank = plsc.cumsum(mi) - mi
            tgt = jnp.where(m, p + rank, ET_PAD + lanes)
            plsc.store_scatter(osrc, [tgt], vsrc[pl.ds(i * 16, 16)])
            plsc.store_scatter(obid, [tgt], vbid[pl.ds(i * 16, 16)])
            return p + plsc.all_reduce_population_count(m)[0]
        pos = lax.fori_loop(0, ET // 16, gbody, pos)
        pos = (pos + B - 1) & jnp.int32(~(B - 1))
    starts[2] = jnp.where(lanes >= C - 32, jnp.full((16,), pos, jnp.int32),
                          starts[2])
    offv[pl.ds(0, 16)] = starts[0]
    offv[pl.ds(16, 16)] = starts[1]
    offv[pl.ds(32, 16)] = starts[2]

    tpad = pl.multiple_of(t * ET_PAD, 8)
    pltpu.sync_copy(osrc.at[pl.ds(0, ET_PAD)], psrc_h.at[pl.ds(tpad, ET_PAD)])
    pltpu.sync_copy(obid.at[pl.ds(0, ET_PAD)], pbid_h.at[pl.ds(tpad, ET_PAD)])
    pltpu.sync_copy(offv, poff_h.at[t])

    for q in range(5):
        r0 = pl.multiple_of(t * 320 + q * 64, 8)
        pltpu.sync_copy(nid_h.at[pl.ds(r0, 64)], idx64)
        pltpu.async_copy(emb_h.at[idx64], rows64, sem).wait()
        pltpu.sync_copy(rows64, h0_h.at[pl.ds(r0, 64), :])


@functools.partial(
    pl.kernel,
    out_type=(
        jax.ShapeDtypeStruct((NW * ET_PAD,), jnp.int32),
        jax.ShapeDtypeStruct((NW * ET_PAD,), jnp.int32),
        jax.ShapeDtypeStruct((NW, 48), jnp.int32),
        jax.ShapeDtypeStruct((N_PAD, D), jnp.float32),
    ),
    mesh=_mesh,
    compiler_params=pltpu.CompilerParams(needs_layout_passes=False),
    scratch_types=[
        pltpu.VMEM((ET,), jnp.int32),
        pltpu.VMEM((ET,), jnp.int32),
        pltpu.VMEM((ET,), jnp.int32),
        pltpu.VMEM((ET,), jnp.int32),
        pltpu.VMEM((ET,), jnp.int32),
        pltpu.VMEM((ET_PAD + 16,), jnp.int32),
        pltpu.VMEM((ET_PAD + 16,), jnp.int32),
        pltpu.VMEM((48,), jnp.int32),
        pltpu.VMEM((64,), jnp.int32),
        pltpu.VMEM((64, D), jnp.float32),
        pltpu.SemaphoreType.DMA,
    ],
)
def _preprocess(*args):
    _pre_body(*args)


def _layer_body(h_h, psrc_h, pbid_h, poff_h, s2_h,
                srcb, bidb, rows0, zbuf,
                offra, offrb, lcnt, cbuf, inv, semg0,
                acc, slab):
    c = lax.axis_index("c")
    s = lax.axis_index("s")
    z16f = jnp.zeros((16,), jnp.float32)
    o16f = jnp.ones((16,), jnp.float32)

    def zb(i, _):
        for k in range(8):
            zbuf[i, pl.ds(k * 16, 16)] = z16f
        return 0
    lax.fori_loop(0, 64, zb, 0)
    pltpu.sync_copy(poff_h.at[2 * s], offra)
    pltpu.sync_copy(poff_h.at[2 * s + 1], offrb)

    def chunk_body(cl, _):
        ci = c * CPS + cl
        for j in range(CHUNK // 64):
            pltpu.sync_copy(zbuf, acc.at[pl.ds(s * CHUNK + j * 64, 64), :])
        def zl_(i, _):
            lcnt[pl.ds(i * 16, 16)] = z16f
            return 0
        lax.fori_loop(0, (NB + 16) // 16, zl_, 0)
        plsc.subcore_barrier()

        for u in range(2):
            offr = offra if u == 0 else offrb
            tt = 2 * s + u
            st = plsc.load_gather(offr, [jnp.full((16,), ci, jnp.int32)])[0]
            en = plsc.load_gather(offr,
                                  [jnp.full((16,), ci + 1, jnp.int32)])[0]
            base = tt * ET_PAD + st
            nb = lax.shift_right_logical(en - st, 5)

            def bbody(bi, _):
                off = pl.multiple_of(base + bi * B, 8)
                pltpu.sync_copy(psrc_h.at[pl.ds(off, B)], srcb)
                pltpu.sync_copy(pbid_h.at[pl.ds(off, B)], bidb)
                pltpu.async_copy(h_h.at[srcb], rows0, semg0).wait()
                pltpu.sync_copy(rows0, acc.at[bidb], add=True)
                for g in range(B // 16):
                    idx16 = bidb[pl.ds(g * 16, 16)]
                    plsc.addupdate_scatter(lcnt, [idx16], o16f)
                return 0
            lax.fori_loop(0, nb, bbody, 0)

        pltpu.sync_copy(lcnt.at[pl.ds(0, NB)], slab.at[s])
        plsc.subcore_barrier()
        pltpu.sync_copy(slab.at[:, pl.ds(s * CHUNK, CHUNK)], cbuf)

        def nvbody(j, _):
            tot = z16f
            for i in range(NS):
                tot = tot + cbuf[i, pl.ds(j * 16, 16)]
            inv[pl.ds(j * 16, 16)] = 1.0 / jnp.maximum(tot, 1.0)
            return 0
        lax.fori_loop(0, CHUNK // 16, nvbody, 0)

        def fbody(fb, _):
            pltpu.sync_copy(acc.at[pl.ds(s * CHUNK + fb * B, B), :], rows0)

            def rowg(gi, _):
                iv = inv[pl.ds(fb * B + gi * 16, 16)]
                for j2 in range(16):
                    sv = iv[j2]
                    for k in range(8):
                        r = gi * 16 + j2
                        rows0[r, pl.ds(k * 16, 16)] = \
                            rows0[r, pl.ds(k * 16, 16)] * sv
                return 0
            lax.fori_loop(0, B // 16, rowg, 0)
            dst_row = ci * NB + s * CHUNK + fb * B
            pltpu.sync_copy(rows0, s2_h.at[pl.ds(dst_row, B), :])
            return 0
        lax.fori_loop(0, CHUNK // B, fbody, 0)
        plsc.subcore_barrier()
        return 0
    lax.fori_loop(0, CPS, chunk_body, 0)


@functools.partial(
    pl.kernel,
    out_type=jax.ShapeDtypeStruct((N_PAD * R, D), jnp.float32),
    mesh=_mesh,
    compiler_params=pltpu.CompilerParams(needs_layout_passes=False),
    scratch_types=[
        pltpu.VMEM((B,), jnp.int32),
        pltpu.VMEM((B,), jnp.int32),
        pltpu.VMEM((B, D), jnp.float32),
        pltpu.VMEM((64, D), jnp.float32),
        pltpu.VMEM((48,), jnp.int32),
        pltpu.VMEM((48,), jnp.int32),
        pltpu.VMEM((NB + 16,), jnp.float32),
        pltpu.VMEM((NS, CHUNK), jnp.float32),
        pltpu.VMEM((CHUNK,), jnp.float32),
        pltpu.SemaphoreType.DMA,
        pltpu.VMEM_SHARED((NB + 8, D), jnp.float32),
        pltpu.VMEM_SHARED((NS, NB), jnp.float32),
    ],
)
def _sc_layer(*args):
    _layer_body(*args)


BN = 256


def _tc_body(act, s3_ref, h_ref, wrel_ref, wself_ref, b_ref, out_ref):
    x3 = s3_ref[:]
    acc = jnp.dot(h_ref[:], wself_ref[:], preferred_element_type=jnp.float32)
    acc = acc + b_ref[:]
    for r in range(R):
        acc = acc + jnp.dot(x3[:, r, :], wrel_ref[r],
                            preferred_element_type=jnp.float32)
    if act:
        acc = jnp.maximum(acc, NEG * acc)
    out_ref[:] = acc


def _tc_layer(s3, h, wrel, wself, bb, act):
    return pl.pallas_call(
        functools.partial(_tc_body, act),
        grid=(N_PAD // BN,),
        in_specs=[
            pl.BlockSpec((BN, R, D), lambda i: (i, 0, 0)),
            pl.BlockSpec((BN, D), lambda i: (i, 0)),
            pl.BlockSpec((R, D, D), lambda i: (0, 0, 0)),
            pl.BlockSpec((D, D), lambda i: (0, 0)),
            pl.BlockSpec((1, D), lambda i: (0, 0)),
        ],
        out_specs=pl.BlockSpec((BN, D), lambda i: (i, 0)),
        out_shape=jax.ShapeDtypeStruct((N_PAD, D), jnp.float32),
    )(s3, h, wrel, wself, bb)


def kernel(emb_table, W_rel, W_self, b, edge_index, edge_type, node_id_copy):
    src = edge_index[0].astype(jnp.int32)
    dst = edge_index[1].astype(jnp.int32)
    typ = edge_type.astype(jnp.int32)
    nid = node_id_copy.astype(jnp.int32)
    nid_pad = jnp.concatenate([nid, jnp.zeros((N_PAD - N,), jnp.int32)])

    psrc, pbid, poff, h = _preprocess(src, dst, typ, nid_pad, emb_table)
    for l in range(3):
        s2 = _sc_layer(h, psrc, pbid, poff)
        s3 = s2.reshape(N_PAD, R, D)
        h = _tc_layer(s3, h, W_rel[l], W_self[l], b[l].reshape(1, D),
                      act=(l < 2))
    return h[:N]
